# baseline (speedup 1.0000x reference)
import functools

import jax
import jax.numpy as jnp
from jax import lax
from jax.experimental import pallas as pl
from jax.experimental.pallas import tpu as pltpu
from jax.experimental.pallas import tpu_sc as plsc

_EXTRA_FRAMES = (1, 3, 5, 7)
_SHARED_TEACHER = (2, 4, 6)
_SHARED_STUDENT = (1, 2, 3)
_TOPK = 4
_EPS = 1e-8


def _dot3(a, b, dims):
    ah = a.astype(jnp.bfloat16)
    al = (a - ah.astype(jnp.float32)).astype(jnp.bfloat16)
    bh = b.astype(jnp.bfloat16)
    bl = (b - bh.astype(jnp.float32)).astype(jnp.bfloat16)
    d = functools.partial(
        lax.dot_general, dimension_numbers=dims,
        preferred_element_type=jnp.float32)
    return d(ah, bl) + d(al, bh) + d(ah, bh)


def _sc_gather_rows(table, idx, per=None):
    n, d = table.shape
    m = idx.shape[0]
    info = plsc.get_sparse_core_info()
    nw = info.num_cores * info.num_subcores
    if per is None:
        per = m // nw
    assert per % 8 == 0 and m % per == 0
    n_active = m // per

    mesh = plsc.VectorSubcoreMesh(core_axis_name="c", subcore_axis_name="s")

    @functools.partial(
        pl.kernel,
        mesh=mesh,
        out_type=jax.ShapeDtypeStruct((m, d), jnp.float32),
        scratch_types=[
            pltpu.VMEM((per,), jnp.int32),
            pltpu.VMEM((per, d), jnp.float32),
            pltpu.SemaphoreType.DMA,
        ],
    )
    def gather_kernel(table_hbm, idx_hbm, out_hbm, idx_v, rows_v, sem):
        wid = lax.axis_index("s") * info.num_cores + lax.axis_index("c")

        @pl.when(wid < n_active)
        def _():
            base = wid * per
            pltpu.sync_copy(idx_hbm.at[pl.ds(base, per)], idx_v)
            pltpu.async_copy(table_hbm.at[idx_v], rows_v, sem).wait()
            pltpu.sync_copy(rows_v, out_hbm.at[pl.ds(base, per)])

    return gather_kernel(table, idx)


def _sc_gather_two(table_a, idx_a, table_b, idx_b):
    d = table_a.shape[1]
    ma, mb = idx_a.shape[0], idx_b.shape[0]
    info = plsc.get_sparse_core_info()
    nw = info.num_cores * info.num_subcores
    per_a, per_b = 16, mb // nw
    assert ma % per_a == 0 and per_b % 8 == 0 and mb % nw == 0
    na = ma // per_a

    mesh = plsc.VectorSubcoreMesh(core_axis_name="c", subcore_axis_name="s")

    @functools.partial(
        pl.kernel,
        mesh=mesh,
        out_type=(jax.ShapeDtypeStruct((ma, d), jnp.float32),
                  jax.ShapeDtypeStruct((mb, d), jnp.float32)),
        scratch_types=[
            pltpu.VMEM((per_a,), jnp.int32),
            pltpu.VMEM((per_a, d), jnp.float32),
            pltpu.VMEM((per_b,), jnp.int32),
            pltpu.VMEM((per_b, d), jnp.float32),
            pltpu.SemaphoreType.DMA,
        ],
    )
    def gather_kernel(ta_hbm, ia_hbm, tb_hbm, ib_hbm, oa_hbm, ob_hbm,
                      ia_v, ra_v, ib_v, rb_v, sem):
        wid = lax.axis_index("s") * info.num_cores + lax.axis_index("c")

        @pl.when(wid < na)
        def _():
            base = wid * per_a
            pltpu.sync_copy(ia_hbm.at[pl.ds(base, per_a)], ia_v)
            pltpu.async_copy(ta_hbm.at[ia_v], ra_v, sem).wait()
            pltpu.sync_copy(ra_v, oa_hbm.at[pl.ds(base, per_a)])

        base_b = wid * per_b
        pltpu.sync_copy(ib_hbm.at[pl.ds(base_b, per_b)], ib_v)
        pltpu.async_copy(tb_hbm.at[ib_v], rb_v, sem).wait()
        pltpu.sync_copy(rb_v, ob_hbm.at[pl.ds(base_b, per_b)])

    return gather_kernel(table_a, idx_a, table_b, idx_b)


def _sim_topk(teacher_feats, reft):
    b_sz, vt, p, d = teacher_feats.shape
    r = reft.shape[1]
    br = b_sz * r
    n_extra = len(_EXTRA_FRAMES)
    fp = n_extra * p

    def body(tf_ref, reft_ref, idx_out, sim_s):
        f = pl.program_id(0)

        cdim = (((1,), (1,)), ((), ()))
        dbf = functools.partial(
            lax.dot_general, dimension_numbers=cdim,
            preferred_element_type=jnp.float32)
        ones_bf = jnp.ones((8, d), jnp.bfloat16)
        for b in range(b_sz):
            x = tf_ref[b, 0]
            xh = x.astype(jnp.bfloat16)
            xl = (x - xh.astype(jnp.float32)).astype(jnp.bfloat16)
            rr = reft_ref[b]
            rh = rr.astype(jnp.bfloat16)
            rl = (rr - rh.astype(jnp.float32)).astype(jnp.bfloat16)
            s = dbf(xh, rl) + dbf(xl, rh) + dbf(xh, rh)
            xnsq = dbf(xh * xh, ones_bf)[:, :1]
            inv = 1.0 / jnp.maximum(jnp.sqrt(xnsq), _EPS)
            sim_s[pl.ds(f * p, p), b * r:(b + 1) * r] = s * inv

        @pl.when(f == n_extra - 1)
        def _():
            sim = sim_s[...]
            iota = lax.broadcasted_iota(jnp.int32, (fp, br), 0)
            bbase = (lax.broadcasted_iota(jnp.int32, (1, br), 1) // r) * (vt * p)
            rows = []
            for _k in range(_TOPK):
                mx = jnp.max(sim, axis=0, keepdims=True)
                cand = jnp.where(sim == mx, iota, jnp.int32(fp))
                pos = jnp.min(cand, axis=0, keepdims=True)
                g = bbase + pos + p * (pos // p + 1)
                rows.append(g)
                sim = jnp.where(iota == pos, -jnp.inf, sim)
            idx_out[...] = jnp.concatenate(rows, axis=0)

    return pl.pallas_call(
        body,
        grid=(n_extra,),
        in_specs=[
            pl.BlockSpec((b_sz, 1, p, d), lambda f: (0, 2 * f + 1, 0, 0)),
            pl.BlockSpec((b_sz, r, d), lambda f: (0, 0, 0)),
        ],
        out_specs=pl.BlockSpec((_TOPK, br), lambda f: (0, 0)),
        out_shape=jax.ShapeDtypeStruct((_TOPK, br), jnp.int32),
        scratch_shapes=[
            pltpu.VMEM((fp, br), jnp.float32),
        ],
    )(teacher_feats, reft)


def _angle_loss(reft, tsh, s_gath, simh, n_iters):
    b_sz, r, d = reft.shape
    k = simh.shape[0]
    rk = k * r
    total_elems = float(n_iters * b_sz * r * r * k)

    dotp = functools.partial(
        lax.dot_general,
        precision=lax.Precision.HIGHEST,
        preferred_element_type=jnp.float32)

    def huber(pred, tgt):
        dd = jnp.abs(pred - tgt)
        return jnp.where(dd < 1.0, 0.5 * dd * dd, dd - 0.5)

    def cosx(dot, usq, vsq):
        return dot / (jnp.maximum(jnp.sqrt(usq), _EPS) *
                      jnp.maximum(jnp.sqrt(vsq), _EPS))

    def body(rt_ref, th_ref, sg_ref, si_ref, out_ref):
        b = pl.program_id(0)
        rt = rt_ref[0]
        th = th_ref[0]
        sg = sg_ref[0]
        si = si_ref[:, 0].reshape(rk, d)

        ones = jnp.ones((1, d), jnp.float32)
        exp_m = (lax.broadcasted_iota(jnp.int32, (r, rk), 0) ==
                 lax.broadcasted_iota(jnp.int32, (r, rk), 1) % r
                 ).astype(jnp.float32)

        cdim = (((1,), (1,)), ((), ()))
        gt = _dot3(th, si, cdim)
        gs = _dot3(sg[r:], si, cdim)
        dg_t = _dot3(rt, si, cdim)
        dg_s = _dot3(sg[:r], si, cdim)
        ht = _dot3(th, rt, cdim)
        hs = _dot3(sg[r:], sg[:r], cdim)

        nsi = _dot3(ones, si * si, cdim)
        nr_t = dotp(_dot3(ones, rt * rt, cdim), exp_m,
                    (((1,), (0,)), ((), ())))
        nr_s = dotp(_dot3(ones, sg[:r] * sg[:r], cdim), exp_m,
                    (((1,), (0,)), ((), ())))
        nsh_t = jnp.sum(th * th, axis=1, keepdims=True)
        nsh_s = jnp.sum(sg[r:] * sg[r:], axis=1, keepdims=True)
        dir_t = jnp.sum(dg_t * exp_m, axis=0, keepdims=True)
        dir_s = jnp.sum(dg_s * exp_m, axis=0, keepdims=True)

        acc = jnp.float32(0.0)
        for i in range(n_iters):
            sl = slice(i * r, (i + 1) * r)
            grids = []
            for gg, hh, nr, nsh, dirx in (
                    (gt, ht, nr_t, nsh_t, dir_t),
                    (gs, hs, nr_s, nsh_s, dir_s)):
                dsi = gg[sl]
                dsr = dotp(hh[i * r:(i + 1) * r], exp_m,
                           (((1,), (0,)), ((), ())))
                ns = nsh[i * r:(i + 1) * r]
                c1 = cosx(dsi - dsr - dirx + nr,
                          ns - 2.0 * dsr + nr, nsi - 2.0 * dirx + nr)
                c2 = cosx(dsr - dirx - dsi + nsi,
                          nr - 2.0 * dirx + nsi, ns - 2.0 * dsi + nsi)
                c3 = cosx(dirx - dsr - dsi + ns,
                          nr - 2.0 * dsr + ns, nsi - 2.0 * dsi + ns)
                grids.append((c1, c2, c3))
            (t1, t2, t3), (p1, p2, p3) = grids
            acc = acc + jnp.sum(huber(p1, t1)) + jnp.sum(huber(p2, t2)) \
                      + jnp.sum(huber(p3, t3))

        @pl.when(b == 0)
        def _():
            out_ref[...] = jnp.zeros((1, 1), jnp.float32)
        scale = jnp.where(b == b_sz - 1, 1.0 / total_elems, 1.0)
        out_ref[...] = (out_ref[...] + jnp.full((1, 1), acc, jnp.float32)) * scale

    return pl.pallas_call(
        body,
        grid=(b_sz,),
        in_specs=[
            pl.BlockSpec((1, r, d), lambda b: (b, 0, 0)),
            pl.BlockSpec((1, n_iters * r, d), lambda b: (b, 0, 0)),
            pl.BlockSpec((1, (1 + n_iters) * r, d), lambda b: (b, 0, 0)),
            pl.BlockSpec((k, 1, r, d), lambda b: (0, b, 0, 0)),
        ],
        out_specs=pl.BlockSpec((1, 1), lambda b: (0, 0)),
        out_shape=jax.ShapeDtypeStruct((1, 1), jnp.float32),
    )(reft, tsh, s_gath, simh)


def kernel(teacher_feats, student_feats, ref_idx, shared_idx):
    b_sz, vt, p, d = teacher_feats.shape
    vs = student_feats.shape[1]
    r = ref_idx.shape[0]
    n_it = len(_SHARED_TEACHER)

    t_flat = teacher_feats.reshape(b_sz * vt * p, d)
    s_flat = student_feats.reshape(b_sz * vs * p, d)

    b_off_t = (jnp.arange(b_sz) * vt * p)[:, None]
    b_off_s = (jnp.arange(b_sz) * vs * p)[:, None]
    tr_idx = (ref_idx[None, :] + b_off_t).reshape(-1)
    th_off = jnp.concatenate([shared_idx + f * p for f in _SHARED_TEACHER])
    th_idx = (th_off[None, :] + b_off_t).reshape(-1)
    s_off = jnp.concatenate(
        [ref_idx] + [shared_idx + f * p for f in _SHARED_STUDENT])
    s_idx = (s_off[None, :] + b_off_s).reshape(-1)

    reft = _sc_gather_rows(t_flat, tr_idx, per=8).reshape(b_sz, r, d)
    tsh, s_gath = _sc_gather_two(t_flat, th_idx, s_flat, s_idx)

    gidx = _sim_topk(teacher_feats, reft)
    simh = _sc_gather_rows(t_flat, gidx.reshape(-1))
    simh = simh.reshape(_TOPK, b_sz, r, d)

    loss = _angle_loss(reft, tsh.reshape(b_sz, n_it * r, d),
                       s_gath.reshape(b_sz, (1 + n_it) * r, d), simh, n_it)
    return loss[0, 0]


# speedup vs baseline: 1.0454x; 1.0454x over previous
import functools

import jax
import jax.numpy as jnp
from jax import lax
from jax.experimental import pallas as pl
from jax.experimental.pallas import tpu as pltpu
from jax.experimental.pallas import tpu_sc as plsc

_EXTRA_FRAMES = (1, 3, 5, 7)
_SHARED_TEACHER = (2, 4, 6)
_SHARED_STUDENT = (1, 2, 3)
_TOPK = 4
_EPS = 1e-8


def _dot3(a, b, dims):
    ah = a.astype(jnp.bfloat16)
    al = (a - ah.astype(jnp.float32)).astype(jnp.bfloat16)
    bh = b.astype(jnp.bfloat16)
    bl = (b - bh.astype(jnp.float32)).astype(jnp.bfloat16)
    d = functools.partial(
        lax.dot_general, dimension_numbers=dims,
        preferred_element_type=jnp.float32)
    return d(ah, bl) + d(al, bh) + d(ah, bh)


def _sc_gather_rows(table, idx, per=None):
    n, d = table.shape
    m = idx.shape[0]
    info = plsc.get_sparse_core_info()
    nw = info.num_cores * info.num_subcores
    if per is None:
        per = m // nw
    assert per % 8 == 0 and m % per == 0
    n_active = m // per

    mesh = plsc.VectorSubcoreMesh(core_axis_name="c", subcore_axis_name="s")

    @functools.partial(
        pl.kernel,
        mesh=mesh,
        out_type=jax.ShapeDtypeStruct((m, d), jnp.float32),
        scratch_types=[
            pltpu.VMEM((per,), jnp.int32),
            pltpu.VMEM((per, d), jnp.float32),
            pltpu.SemaphoreType.DMA,
        ],
    )
    def gather_kernel(table_hbm, idx_hbm, out_hbm, idx_v, rows_v, sem):
        wid = lax.axis_index("s") * info.num_cores + lax.axis_index("c")

        @pl.when(wid < n_active)
        def _():
            base = wid * per
            pltpu.sync_copy(idx_hbm.at[pl.ds(base, per)], idx_v)
            pltpu.async_copy(table_hbm.at[idx_v], rows_v, sem).wait()
            pltpu.sync_copy(rows_v, out_hbm.at[pl.ds(base, per)])

    return gather_kernel(table, idx)


def _sc_gather_two(table_a, idx_a, table_b, idx_b):
    d = table_a.shape[1]
    ma, mb = idx_a.shape[0], idx_b.shape[0]
    info = plsc.get_sparse_core_info()
    nw = info.num_cores * info.num_subcores
    per_a, per_b = 16, mb // nw
    assert ma % per_a == 0 and per_b % 8 == 0 and mb % nw == 0
    na = ma // per_a

    mesh = plsc.VectorSubcoreMesh(core_axis_name="c", subcore_axis_name="s")

    @functools.partial(
        pl.kernel,
        mesh=mesh,
        out_type=(jax.ShapeDtypeStruct((ma, d), jnp.float32),
                  jax.ShapeDtypeStruct((mb, d), jnp.float32)),
        scratch_types=[
            pltpu.VMEM((per_a,), jnp.int32),
            pltpu.VMEM((per_a, d), jnp.float32),
            pltpu.VMEM((per_b,), jnp.int32),
            pltpu.VMEM((per_b, d), jnp.float32),
            pltpu.SemaphoreType.DMA,
        ],
    )
    def gather_kernel(ta_hbm, ia_hbm, tb_hbm, ib_hbm, oa_hbm, ob_hbm,
                      ia_v, ra_v, ib_v, rb_v, sem):
        wid = lax.axis_index("s") * info.num_cores + lax.axis_index("c")

        @pl.when(wid < na)
        def _():
            base = wid * per_a
            pltpu.sync_copy(ia_hbm.at[pl.ds(base, per_a)], ia_v)
            pltpu.async_copy(ta_hbm.at[ia_v], ra_v, sem).wait()
            pltpu.sync_copy(ra_v, oa_hbm.at[pl.ds(base, per_a)])

        base_b = wid * per_b
        pltpu.sync_copy(ib_hbm.at[pl.ds(base_b, per_b)], ib_v)
        pltpu.async_copy(tb_hbm.at[ib_v], rb_v, sem).wait()
        pltpu.sync_copy(rb_v, ob_hbm.at[pl.ds(base_b, per_b)])

    return gather_kernel(table_a, idx_a, table_b, idx_b)


def _sim_topk(teacher_feats, reft):
    b_sz, vt, p, d = teacher_feats.shape
    r = reft.shape[1]
    br = b_sz * r
    n_extra = len(_EXTRA_FRAMES)
    fp = n_extra * p

    def body(tf_ref, reft_ref, idx_out, sim_s):
        f = pl.program_id(0)

        for b in range(b_sz):
            x = tf_ref[b, 0]
            xnsq = jnp.sum(x * x, axis=1, keepdims=True)
            s = _dot3(x, reft_ref[b], (((1,), (1,)), ((), ())))
            inv = 1.0 / jnp.maximum(jnp.sqrt(xnsq), _EPS)
            sim_s[pl.ds(f * p, p), b * r:(b + 1) * r] = s * inv

        @pl.when(f == n_extra - 1)
        def _():
            sim = sim_s[...]
            iota = lax.broadcasted_iota(jnp.int32, (fp, br), 0)
            bbase = (lax.broadcasted_iota(jnp.int32, (1, br), 1) // r) * (vt * p)
            rows = []
            for _k in range(_TOPK):
                mx = jnp.max(sim, axis=0, keepdims=True)
                cand = jnp.where(sim == mx, iota, jnp.int32(fp))
                pos = jnp.min(cand, axis=0, keepdims=True)
                g = bbase + pos + p * (pos // p + 1)
                rows.append(g)
                sim = jnp.where(iota == pos, -jnp.inf, sim)
            idx_out[...] = jnp.concatenate(rows, axis=0)

    return pl.pallas_call(
        body,
        grid=(n_extra,),
        in_specs=[
            pl.BlockSpec((b_sz, 1, p, d), lambda f: (0, 2 * f + 1, 0, 0)),
            pl.BlockSpec((b_sz, r, d), lambda f: (0, 0, 0)),
        ],
        out_specs=pl.BlockSpec((_TOPK, br), lambda f: (0, 0)),
        out_shape=jax.ShapeDtypeStruct((_TOPK, br), jnp.int32),
        scratch_shapes=[
            pltpu.VMEM((fp, br), jnp.float32),
        ],
    )(teacher_feats, reft)


def _angle_loss(reft, tsh, s_gath, simh, n_iters):
    b_sz, r, d = reft.shape
    k = simh.shape[0]
    rk = k * r
    total_elems = float(n_iters * b_sz * r * r * k)

    dotp = functools.partial(
        lax.dot_general,
        precision=lax.Precision.HIGHEST,
        preferred_element_type=jnp.float32)

    def huber(pred, tgt):
        dd = jnp.abs(pred - tgt)
        return jnp.where(dd < 1.0, 0.5 * dd * dd, dd - 0.5)

    def cosx(dot, usq, vsq):
        return dot / (jnp.maximum(jnp.sqrt(usq), _EPS) *
                      jnp.maximum(jnp.sqrt(vsq), _EPS))

    def body(rt_ref, th_ref, sg_ref, si_ref, exp_ref, out_ref):
        b = pl.program_id(0)
        rt = rt_ref[0]
        th = th_ref[0]
        sg = sg_ref[0]
        si = si_ref[:, 0].reshape(rk, d)

        ones = jnp.ones((1, d), jnp.float32)
        exp_m = exp_ref[...]

        cdim = (((1,), (1,)), ((), ()))
        gt = _dot3(th, si, cdim)
        gs = _dot3(sg[r:], si, cdim)
        dg_t = _dot3(rt, si, cdim)
        dg_s = _dot3(sg[:r], si, cdim)
        ht = _dot3(th, rt, cdim)
        hs = _dot3(sg[r:], sg[:r], cdim)

        nsi = _dot3(ones, si * si, cdim)
        nr_t = dotp(_dot3(ones, rt * rt, cdim), exp_m,
                    (((1,), (0,)), ((), ())))
        nr_s = dotp(_dot3(ones, sg[:r] * sg[:r], cdim), exp_m,
                    (((1,), (0,)), ((), ())))
        nsh_t = jnp.sum(th * th, axis=1, keepdims=True)
        nsh_s = jnp.sum(sg[r:] * sg[r:], axis=1, keepdims=True)
        dir_t = jnp.sum(dg_t * exp_m, axis=0, keepdims=True)
        dir_s = jnp.sum(dg_s * exp_m, axis=0, keepdims=True)

        acc = jnp.float32(0.0)
        for i in range(n_iters):
            sl = slice(i * r, (i + 1) * r)
            grids = []
            for gg, hh, nr, nsh, dirx in (
                    (gt, ht, nr_t, nsh_t, dir_t),
                    (gs, hs, nr_s, nsh_s, dir_s)):
                dsi = gg[sl]
                dsr = dotp(hh[i * r:(i + 1) * r], exp_m,
                           (((1,), (0,)), ((), ())))
                ns = nsh[i * r:(i + 1) * r]
                c1 = cosx(dsi - dsr - dirx + nr,
                          ns - 2.0 * dsr + nr, nsi - 2.0 * dirx + nr)
                c2 = cosx(dsr - dirx - dsi + nsi,
                          nr - 2.0 * dirx + nsi, ns - 2.0 * dsi + nsi)
                c3 = cosx(dirx - dsr - dsi + ns,
                          nr - 2.0 * dsr + ns, nsi - 2.0 * dsi + ns)
                grids.append((c1, c2, c3))
            (t1, t2, t3), (p1, p2, p3) = grids
            acc = acc + jnp.sum(huber(p1, t1)) + jnp.sum(huber(p2, t2)) \
                      + jnp.sum(huber(p3, t3))

        @pl.when(b == 0)
        def _():
            out_ref[...] = jnp.zeros((1, 1), jnp.float32)
        scale = jnp.where(b == b_sz - 1, 1.0 / total_elems, 1.0)
        out_ref[...] = (out_ref[...] + jnp.full((1, 1), acc, jnp.float32)) * scale

    return pl.pallas_call(
        body,
        grid=(b_sz,),
        in_specs=[
            pl.BlockSpec((1, r, d), lambda b: (b, 0, 0)),
            pl.BlockSpec((1, n_iters * r, d), lambda b: (b, 0, 0)),
            pl.BlockSpec((1, (1 + n_iters) * r, d), lambda b: (b, 0, 0)),
            pl.BlockSpec((k, 1, r, d), lambda b: (0, b, 0, 0)),
            pl.BlockSpec((r, rk), lambda b: (0, 0)),
        ],
        out_specs=pl.BlockSpec((1, 1), lambda b: (0, 0)),
        out_shape=jax.ShapeDtypeStruct((1, 1), jnp.float32),
    )(reft, tsh, s_gath, simh,
      (jnp.arange(r, dtype=jnp.int32)[:, None] ==
       jnp.arange(rk, dtype=jnp.int32)[None, :] % r).astype(jnp.float32))


def kernel(teacher_feats, student_feats, ref_idx, shared_idx):
    b_sz, vt, p, d = teacher_feats.shape
    vs = student_feats.shape[1]
    r = ref_idx.shape[0]
    n_it = len(_SHARED_TEACHER)

    t_flat = teacher_feats.reshape(b_sz * vt * p, d)
    s_flat = student_feats.reshape(b_sz * vs * p, d)

    b_off_t = (jnp.arange(b_sz) * vt * p)[:, None]
    b_off_s = (jnp.arange(b_sz) * vs * p)[:, None]
    tr_idx = (ref_idx[None, :] + b_off_t).reshape(-1)
    th_off = jnp.concatenate([shared_idx + f * p for f in _SHARED_TEACHER])
    th_idx = (th_off[None, :] + b_off_t).reshape(-1)
    s_off = jnp.concatenate(
        [ref_idx] + [shared_idx + f * p for f in _SHARED_STUDENT])
    s_idx = (s_off[None, :] + b_off_s).reshape(-1)

    reft = _sc_gather_rows(t_flat, tr_idx, per=8).reshape(b_sz, r, d)
    tsh, s_gath = _sc_gather_two(t_flat, th_idx, s_flat, s_idx)

    gidx = _sim_topk(teacher_feats, reft)
    simh = _sc_gather_rows(t_flat, gidx.reshape(-1))
    simh = simh.reshape(_TOPK, b_sz, r, d)

    loss = _angle_loss(reft, tsh.reshape(b_sz, n_it * r, d),
                       s_gath.reshape(b_sz, (1 + n_it) * r, d), simh, n_it)
    return loss[0, 0]


# speedup vs baseline: 1.0814x; 1.0344x over previous
import functools

import jax
import jax.numpy as jnp
from jax import lax
from jax.experimental import pallas as pl
from jax.experimental.pallas import tpu as pltpu
from jax.experimental.pallas import tpu_sc as plsc

_EXTRA_FRAMES = (1, 3, 5, 7)
_SHARED_TEACHER = (2, 4, 6)
_SHARED_STUDENT = (1, 2, 3)
_TOPK = 4
_EPS = 1e-8


def _dot3(a, b, dims):
    ah = a.astype(jnp.bfloat16)
    al = (a - ah.astype(jnp.float32)).astype(jnp.bfloat16)
    bh = b.astype(jnp.bfloat16)
    bl = (b - bh.astype(jnp.float32)).astype(jnp.bfloat16)
    d = functools.partial(
        lax.dot_general, dimension_numbers=dims,
        preferred_element_type=jnp.float32)
    return d(ah, bl) + d(al, bh) + d(ah, bh)


def _sc_gather_rows(table, idx, per=None):
    n, d = table.shape
    m = idx.shape[0]
    info = plsc.get_sparse_core_info()
    nw = info.num_cores * info.num_subcores
    if per is None:
        per = m // nw
    assert per % 8 == 0 and m % per == 0
    n_active = m // per

    mesh = plsc.VectorSubcoreMesh(core_axis_name="c", subcore_axis_name="s")

    @functools.partial(
        pl.kernel,
        mesh=mesh,
        out_type=jax.ShapeDtypeStruct((m, d), jnp.float32),
        scratch_types=[
            pltpu.VMEM((per,), jnp.int32),
            pltpu.VMEM((per, d), jnp.float32),
            pltpu.SemaphoreType.DMA,
        ],
    )
    def gather_kernel(table_hbm, idx_hbm, out_hbm, idx_v, rows_v, sem):
        wid = lax.axis_index("s") * info.num_cores + lax.axis_index("c")

        @pl.when(wid < n_active)
        def _():
            base = wid * per
            pltpu.sync_copy(idx_hbm.at[pl.ds(base, per)], idx_v)
            pltpu.async_copy(table_hbm.at[idx_v], rows_v, sem).wait()
            pltpu.sync_copy(rows_v, out_hbm.at[pl.ds(base, per)])

    return gather_kernel(table, idx)


def _sc_gather_two(table_a, idx_a, table_b, idx_b):
    d = table_a.shape[1]
    ma, mb = idx_a.shape[0], idx_b.shape[0]
    info = plsc.get_sparse_core_info()
    nw = info.num_cores * info.num_subcores
    per_a, per_b = 16, mb // nw
    assert ma % per_a == 0 and per_b % 8 == 0 and mb % nw == 0
    na = ma // per_a

    mesh = plsc.VectorSubcoreMesh(core_axis_name="c", subcore_axis_name="s")

    @functools.partial(
        pl.kernel,
        mesh=mesh,
        out_type=(jax.ShapeDtypeStruct((ma, d), jnp.float32),
                  jax.ShapeDtypeStruct((mb, d), jnp.float32)),
        scratch_types=[
            pltpu.VMEM((per_a,), jnp.int32),
            pltpu.VMEM((per_a, d), jnp.float32),
            pltpu.VMEM((per_b,), jnp.int32),
            pltpu.VMEM((per_b, d), jnp.float32),
            pltpu.SemaphoreType.DMA,
        ],
    )
    def gather_kernel(ta_hbm, ia_hbm, tb_hbm, ib_hbm, oa_hbm, ob_hbm,
                      ia_v, ra_v, ib_v, rb_v, sem):
        wid = lax.axis_index("s") * info.num_cores + lax.axis_index("c")

        @pl.when(wid < na)
        def _():
            base = wid * per_a
            pltpu.sync_copy(ia_hbm.at[pl.ds(base, per_a)], ia_v)
            pltpu.async_copy(ta_hbm.at[ia_v], ra_v, sem).wait()
            pltpu.sync_copy(ra_v, oa_hbm.at[pl.ds(base, per_a)])

        base_b = wid * per_b
        pltpu.sync_copy(ib_hbm.at[pl.ds(base_b, per_b)], ib_v)
        pltpu.async_copy(tb_hbm.at[ib_v], rb_v, sem).wait()
        pltpu.sync_copy(rb_v, ob_hbm.at[pl.ds(base_b, per_b)])

    return gather_kernel(table_a, idx_a, table_b, idx_b)


def _sim_topk(teacher_feats, reft):
    b_sz, vt, p, d = teacher_feats.shape
    r = reft.shape[1]
    br = b_sz * r
    n_extra = len(_EXTRA_FRAMES)
    fp = n_extra * p

    def body(tf_ref, reft_ref, idx_out, sim_s):
        f = pl.program_id(0)

        cdim = (((1,), (1,)), ((), ()))
        dbf = functools.partial(
            lax.dot_general, dimension_numbers=cdim,
            preferred_element_type=jnp.float32)
        for b in range(b_sz):
            x = tf_ref[b, 0]
            xnsq = jnp.sum(x * x, axis=1, keepdims=True)
            xh = x.astype(jnp.bfloat16)
            xl = (x - xh.astype(jnp.float32)).astype(jnp.bfloat16)
            rh = reft_ref[b].astype(jnp.bfloat16)
            s = dbf(xh, rh) + dbf(xl, rh)
            inv = 1.0 / jnp.maximum(jnp.sqrt(xnsq), _EPS)
            sim_s[pl.ds(f * p, p), b * r:(b + 1) * r] = s * inv

        @pl.when(f == n_extra - 1)
        def _():
            sim = sim_s[...]
            iota = lax.broadcasted_iota(jnp.int32, (fp, br), 0)
            bbase = (lax.broadcasted_iota(jnp.int32, (1, br), 1) // r) * (vt * p)
            rows = []
            for _k in range(_TOPK):
                mx = jnp.max(sim, axis=0, keepdims=True)
                cand = jnp.where(sim == mx, iota, jnp.int32(fp))
                pos = jnp.min(cand, axis=0, keepdims=True)
                g = bbase + pos + p * (pos // p + 1)
                rows.append(g)
                sim = jnp.where(iota == pos, -jnp.inf, sim)
            idx_out[...] = jnp.concatenate(rows, axis=0)

    return pl.pallas_call(
        body,
        grid=(n_extra,),
        in_specs=[
            pl.BlockSpec((b_sz, 1, p, d), lambda f: (0, 2 * f + 1, 0, 0)),
            pl.BlockSpec((b_sz, r, d), lambda f: (0, 0, 0)),
        ],
        out_specs=pl.BlockSpec((_TOPK, br), lambda f: (0, 0)),
        out_shape=jax.ShapeDtypeStruct((_TOPK, br), jnp.int32),
        scratch_shapes=[
            pltpu.VMEM((fp, br), jnp.float32),
        ],
    )(teacher_feats, reft)


def _angle_loss(reft, tsh, s_gath, simh, n_iters):
    b_sz, r, d = reft.shape
    k = simh.shape[0]
    rk = k * r
    total_elems = float(n_iters * b_sz * r * r * k)

    dotp = functools.partial(
        lax.dot_general,
        precision=lax.Precision.HIGHEST,
        preferred_element_type=jnp.float32)

    def huber(pred, tgt):
        dd = jnp.abs(pred - tgt)
        return jnp.where(dd < 1.0, 0.5 * dd * dd, dd - 0.5)

    def cosx(dot, usq, vsq):
        return dot / (jnp.maximum(jnp.sqrt(usq), _EPS) *
                      jnp.maximum(jnp.sqrt(vsq), _EPS))

    def body(rt_ref, th_ref, sg_ref, si_ref, exp_ref, out_ref):
        b = pl.program_id(0)
        rt = rt_ref[0]
        th = th_ref[0]
        sg = sg_ref[0]
        si = si_ref[:, 0].reshape(rk, d)

        ones = jnp.ones((1, d), jnp.float32)
        exp_m = exp_ref[...]

        cdim = (((1,), (1,)), ((), ()))
        gt = _dot3(th, si, cdim)
        gs = _dot3(sg[r:], si, cdim)
        dg_t = _dot3(rt, si, cdim)
        dg_s = _dot3(sg[:r], si, cdim)
        ht = _dot3(th, rt, cdim)
        hs = _dot3(sg[r:], sg[:r], cdim)

        nsi = _dot3(ones, si * si, cdim)
        nr_t = dotp(_dot3(ones, rt * rt, cdim), exp_m,
                    (((1,), (0,)), ((), ())))
        nr_s = dotp(_dot3(ones, sg[:r] * sg[:r], cdim), exp_m,
                    (((1,), (0,)), ((), ())))
        nsh_t = jnp.sum(th * th, axis=1, keepdims=True)
        nsh_s = jnp.sum(sg[r:] * sg[r:], axis=1, keepdims=True)
        dir_t = jnp.sum(dg_t * exp_m, axis=0, keepdims=True)
        dir_s = jnp.sum(dg_s * exp_m, axis=0, keepdims=True)

        acc = jnp.float32(0.0)
        for i in range(n_iters):
            sl = slice(i * r, (i + 1) * r)
            grids = []
            for gg, hh, nr, nsh, dirx in (
                    (gt, ht, nr_t, nsh_t, dir_t),
                    (gs, hs, nr_s, nsh_s, dir_s)):
                dsi = gg[sl]
                dsr = dotp(hh[i * r:(i + 1) * r], exp_m,
                           (((1,), (0,)), ((), ())))
                ns = nsh[i * r:(i + 1) * r]
                c1 = cosx(dsi - dsr - dirx + nr,
                          ns - 2.0 * dsr + nr, nsi - 2.0 * dirx + nr)
                c2 = cosx(dsr - dirx - dsi + nsi,
                          nr - 2.0 * dirx + nsi, ns - 2.0 * dsi + nsi)
                c3 = cosx(dirx - dsr - dsi + ns,
                          nr - 2.0 * dsr + ns, nsi - 2.0 * dsi + ns)
                grids.append((c1, c2, c3))
            (t1, t2, t3), (p1, p2, p3) = grids
            acc = acc + jnp.sum(huber(p1, t1)) + jnp.sum(huber(p2, t2)) \
                      + jnp.sum(huber(p3, t3))

        @pl.when(b == 0)
        def _():
            out_ref[...] = jnp.zeros((1, 1), jnp.float32)
        scale = jnp.where(b == b_sz - 1, 1.0 / total_elems, 1.0)
        out_ref[...] = (out_ref[...] + jnp.full((1, 1), acc, jnp.float32)) * scale

    return pl.pallas_call(
        body,
        grid=(b_sz,),
        in_specs=[
            pl.BlockSpec((1, r, d), lambda b: (b, 0, 0)),
            pl.BlockSpec((1, n_iters * r, d), lambda b: (b, 0, 0)),
            pl.BlockSpec((1, (1 + n_iters) * r, d), lambda b: (b, 0, 0)),
            pl.BlockSpec((k, 1, r, d), lambda b: (0, b, 0, 0)),
            pl.BlockSpec((r, rk), lambda b: (0, 0)),
        ],
        out_specs=pl.BlockSpec((1, 1), lambda b: (0, 0)),
        out_shape=jax.ShapeDtypeStruct((1, 1), jnp.float32),
    )(reft, tsh, s_gath, simh,
      (jnp.arange(r, dtype=jnp.int32)[:, None] ==
       jnp.arange(rk, dtype=jnp.int32)[None, :] % r).astype(jnp.float32))


def kernel(teacher_feats, student_feats, ref_idx, shared_idx):
    b_sz, vt, p, d = teacher_feats.shape
    vs = student_feats.shape[1]
    r = ref_idx.shape[0]
    n_it = len(_SHARED_TEACHER)

    t_flat = teacher_feats.reshape(b_sz * vt * p, d)
    s_flat = student_feats.reshape(b_sz * vs * p, d)

    b_off_t = (jnp.arange(b_sz) * vt * p)[:, None]
    b_off_s = (jnp.arange(b_sz) * vs * p)[:, None]
    tr_idx = (ref_idx[None, :] + b_off_t).reshape(-1)
    th_off = jnp.concatenate([shared_idx + f * p for f in _SHARED_TEACHER])
    th_idx = (th_off[None, :] + b_off_t).reshape(-1)
    s_off = jnp.concatenate(
        [ref_idx] + [shared_idx + f * p for f in _SHARED_STUDENT])
    s_idx = (s_off[None, :] + b_off_s).reshape(-1)

    reft = _sc_gather_rows(t_flat, tr_idx, per=8).reshape(b_sz, r, d)
    tsh, s_gath = _sc_gather_two(t_flat, th_idx, s_flat, s_idx)

    gidx = _sim_topk(teacher_feats, reft)
    simh = _sc_gather_rows(t_flat, gidx.reshape(-1))
    simh = simh.reshape(_TOPK, b_sz, r, d)

    loss = _angle_loss(reft, tsh.reshape(b_sz, n_it * r, d),
                       s_gath.reshape(b_sz, (1 + n_it) * r, d), simh, n_it)
    return loss[0, 0]
